# Initial kernel scaffold; baseline (speedup 1.0000x reference)
import jax
import jax.numpy as jnp
from jax.experimental import pallas as pl


def kernel(Z, S, D):
    raise NotImplementedError("write your pallas kernel here")


# speedup vs baseline: 3.1580x; 3.1580x over previous
import functools

import jax
import jax.numpy as jnp
import numpy as np
from jax import lax
from jax.experimental import pallas as pl
from jax.experimental.pallas import tpu as pltpu
from jax.experimental.pallas import tpu_sc as plsc

_LOG_2PI = float(np.log(2.0 * np.pi))

_NC = 2
_NS = 16
_NW = _NC * _NS
_CHUNK = 8


def _softmax_qu(z, s):
    u = 0.5 * (z * z + s * s) + _LOG_2PI
    nu = -u
    m = jnp.max(nu, axis=-1, keepdims=True)
    e = jnp.exp(nu - m)
    q = e / jnp.sum(e, axis=-1, keepdims=True)
    return q, u


def _qc_body(z_ref, s_ref, qc_ref):
    q, _ = _softmax_qu(z_ref[...], s_ref[...])
    qc_ref[...] = jnp.clip(q, 1e-6, 1 - 1e-6)


def _final_body(z_ref, s_ref, d_ref, co_ref, out_ref):
    q, u = _softmax_qu(z_ref[...], s_ref[...])
    e_pt = jnp.sum(q * u, axis=-1)
    pot = -jnp.log(co_ref[...])
    pot = jnp.where(d_ref[...] < 0, 0.0, pot)
    out_ref[...] = e_pt + jnp.sum(pot, axis=-1)


def _sc_body(n_nodes, deg, qc_hbm, d_hbm, co_hbm, idx_v, neigh_v, qrows_v,
             co_v, sem):
    cid = lax.axis_index("c")
    sid = lax.axis_index("s")
    wid = sid * _NC + cid
    n_chunks = n_nodes // _CHUNK
    edges = _CHUNK * deg
    nrow = edges // 128
    lo = wid * n_chunks // _NW
    hi = (wid + 1) * n_chunks // _NW

    def chunk_body(g, carry):
        node_base = g * _CHUNK
        pltpu.sync_copy(d_hbm.at[pl.ds(g * nrow, nrow)], idx_v)
        pltpu.sync_copy(qc_hbm.at[pl.ds(node_base, _CHUNK)], qrows_v)
        cps = [
            pltpu.async_copy(qc_hbm.at[idx_v.at[j]],
                             neigh_v.at[pl.ds(j * 128, 128)], sem)
            for j in range(nrow)
        ]
        for cp in cps:
            cp.wait()
        lanes = lax.iota(jnp.int32, 16)
        gdn = lax.GatherDimensionNumbers(
            offset_dims=(), collapsed_slice_dims=(0,), start_index_map=(0,))
        perms = [(lanes ^ sh).reshape(16, 1) for sh in (1, 2, 4, 8)]
        for i in range(_CHUNK):
            qs = [qrows_v[i, pl.ds(16 * t, 16)] for t in range(8)]
            for h in range(deg // 16):
                base = i * deg + h * 16

                def edge_body(j, acc, base=base, qs=qs):
                    e = base + j
                    p = neigh_v[e, pl.ds(0, 16)] * qs[0]
                    for t in range(1, 8):
                        p = p + neigh_v[e, pl.ds(16 * t, 16)] * qs[t]
                    for perm in perms:
                        p = p + lax.gather(
                            p, perm, gdn, (1,),
                            mode=lax.GatherScatterMode.PROMISE_IN_BOUNDS)
                    return jnp.where(lanes == j, p, acc)

                acc = lax.fori_loop(0, 16, edge_body,
                                    jnp.zeros((16,), jnp.float32))
                co_v[pl.ds(base, 16)] = acc
        pltpu.sync_copy(co_v, co_hbm.at[pl.ds(g * edges, edges)])
        return carry

    lax.fori_loop(lo, hi, chunk_body, 0)


def kernel(Z, S, D):
    n, k = Z.shape
    deg = D.shape[1]
    assert k == 128 and n % _CHUNK == 0 and (n * deg) % 128 == 0

    blk = 2000
    grid = n // blk
    qc = pl.pallas_call(
        _qc_body,
        grid=(grid,),
        in_specs=[
            pl.BlockSpec((blk, k), lambda i: (i, 0)),
            pl.BlockSpec((blk, k), lambda i: (i, 0)),
        ],
        out_specs=pl.BlockSpec((blk, k), lambda i: (i, 0)),
        out_shape=jax.ShapeDtypeStruct((n, k), jnp.float32),
    )(Z, S)

    d_rows = D.reshape(n * deg // 128, 128)
    edges = _CHUNK * deg
    sc = functools.partial(
        pl.kernel,
        out_type=jax.ShapeDtypeStruct((n * deg,), jnp.float32),
        mesh=plsc.VectorSubcoreMesh(core_axis_name="c", subcore_axis_name="s"),
        scratch_types=[
            pltpu.VMEM((edges // 128, 128), jnp.int32),
            pltpu.VMEM((edges, k), jnp.float32),
            pltpu.VMEM((_CHUNK, k), jnp.float32),
            pltpu.VMEM((edges,), jnp.float32),
            pltpu.SemaphoreType.DMA,
        ],
    )(functools.partial(_sc_body, n, deg))
    co = sc(qc, d_rows).reshape(n, deg)

    energy = pl.pallas_call(
        _final_body,
        out_shape=jax.ShapeDtypeStruct((n,), jnp.float32),
    )(Z, S, D, co)
    return energy


# speedup vs baseline: 3.5405x; 1.1211x over previous
import functools

import jax
import jax.numpy as jnp
import numpy as np
from jax import lax
from jax.experimental import pallas as pl
from jax.experimental.pallas import tpu as pltpu
from jax.experimental.pallas import tpu_sc as plsc

_LOG_2PI = float(np.log(2.0 * np.pi))

_NC = 2
_NS = 16
_NW = _NC * _NS
_CHUNK = 8


def _softmax_qu(z, s):
    u = 0.5 * (z * z + s * s) + _LOG_2PI
    nu = -u
    m = jnp.max(nu, axis=-1, keepdims=True)
    e = jnp.exp(nu - m)
    q = e / jnp.sum(e, axis=-1, keepdims=True)
    return q, u


def _qc_body(z_ref, s_ref, qc_ref):
    q, _ = _softmax_qu(z_ref[...], s_ref[...])
    qc_ref[...] = jnp.clip(q, 1e-6, 1 - 1e-6)


def _final_body(z_ref, s_ref, d_ref, co_ref, out_ref):
    q, u = _softmax_qu(z_ref[...], s_ref[...])
    e_pt = jnp.sum(q * u, axis=-1)
    pot = -jnp.log(co_ref[...])
    pot = jnp.where(d_ref[...] < 0, 0.0, pot)
    out_ref[...] = e_pt + jnp.sum(pot, axis=-1)


def _sc_body(n_nodes, deg, qc_hbm, d_hbm, co_hbm, idx_v, q_v, neigh0, neigh1,
             co_v, sg0, sg1):
    cid = lax.axis_index("c")
    sid = lax.axis_index("s")
    wid = sid * _NC + cid
    edges = _CHUNK * deg
    n_chunks = n_nodes // _CHUNK
    npt = n_chunks // 2
    lo_p = wid * npt // _NW
    hi_p = (wid + 1) * npt // _NW
    npairs = hi_p - lo_p
    max_pairs = (npt + _NW - 1) // _NW
    lo_r = (lo_p // 2) * 8
    off = lo_p * 4 - lo_r
    pltpu.sync_copy(d_hbm.at[pl.ds(lo_r, max_pairs * 4 + 8)], idx_v)
    pltpu.sync_copy(qc_hbm.at[pl.ds(lo_p * 2 * _CHUNK, max_pairs * 2 * _CHUNK)],
                    q_v)

    lanes = lax.iota(jnp.int32, 16)
    gdn = lax.GatherDimensionNumbers(
        offset_dims=(), collapsed_slice_dims=(0,), start_index_map=(0,))
    perms = [(lanes ^ sh).reshape(16, 1) for sh in (1, 2, 4, 8)]

    def issue(row0, nbuf, sem):
        pltpu.async_copy(qc_hbm.at[idx_v.at[off + row0]],
                         nbuf.at[pl.ds(0, 128)], sem)
        pltpu.async_copy(qc_hbm.at[idx_v.at[off + row0 + 1]],
                         nbuf.at[pl.ds(128, 128)], sem)

    def drain(nbuf, sem):
        pltpu.make_async_copy(qc_hbm.at[idx_v.at[0]],
                              nbuf.at[pl.ds(0, 128)], sem).wait()
        pltpu.make_async_copy(qc_hbm.at[idx_v.at[0]],
                              nbuf.at[pl.ds(128, 128)], sem).wait()

    def compute(cl, nbuf):
        for i in range(_CHUNK):
            nloc = cl * _CHUNK + i
            qs = [q_v[nloc, pl.ds(16 * t, 16)] for t in range(8)]
            for h in range(deg // 16):
                base = i * deg + h * 16

                def edge_body(j, acc, base=base, qs=qs):
                    e = base + j
                    p = nbuf[e, pl.ds(0, 16)] * qs[0]
                    for t in range(1, 8):
                        p = p + nbuf[e, pl.ds(16 * t, 16)] * qs[t]
                    for perm in perms:
                        p = p + lax.gather(
                            p, perm, gdn, (1,),
                            mode=lax.GatherScatterMode.PROMISE_IN_BOUNDS)
                    return jnp.where(lanes == j, p, acc)

                acc = lax.fori_loop(0, 16, edge_body,
                                    jnp.zeros((16,), jnp.float32), unroll=4)
                co_v[pl.ds(cl * edges + base, 16)] = acc

    issue(0, neigh0, sg0)

    def pair_body(p, carry):
        r = p * 4
        issue(r + 2, neigh1, sg1)
        drain(neigh0, sg0)
        compute(2 * p, neigh0)

        @pl.when(p + 1 < npairs)
        def _():
            issue(r + 4, neigh0, sg0)

        drain(neigh1, sg1)
        compute(2 * p + 1, neigh1)
        return carry

    lax.fori_loop(0, npairs, pair_body, 0)
    body_e = (max_pairs - 1) * 2 * edges
    pltpu.sync_copy(co_v.at[pl.ds(0, body_e)],
                    co_hbm.at[pl.ds(lo_p * 2 * edges, body_e)])

    @pl.when(npairs == max_pairs)
    def _():
        pltpu.sync_copy(
            co_v.at[pl.ds(body_e, 2 * edges)],
            co_hbm.at[pl.ds(lo_p * 2 * edges + body_e, 2 * edges)])


def kernel(Z, S, D):
    n, k = Z.shape
    deg = D.shape[1]
    assert k == 128 and n % _CHUNK == 0 and (n * deg) % 128 == 0

    blk = 2000
    grid = n // blk
    qc = pl.pallas_call(
        _qc_body,
        grid=(grid,),
        in_specs=[
            pl.BlockSpec((blk, k), lambda i: (i, 0)),
            pl.BlockSpec((blk, k), lambda i: (i, 0)),
        ],
        out_specs=pl.BlockSpec((blk, k), lambda i: (i, 0)),
        out_shape=jax.ShapeDtypeStruct((n, k), jnp.float32),
    )(Z, S)

    d_rows = jnp.pad(D.reshape(n * deg // 128, 128), ((0, 8), (0, 0)))
    edges = _CHUNK * deg
    npt = (n // _CHUNK) // 2
    max_pairs = (npt + _NW - 1) // _NW
    sc = functools.partial(
        pl.kernel,
        out_type=jax.ShapeDtypeStruct((n * deg,), jnp.float32),
        mesh=plsc.VectorSubcoreMesh(core_axis_name="c", subcore_axis_name="s"),
        scratch_types=[
            pltpu.VMEM((max_pairs * 4 + 8, 128), jnp.int32),
            pltpu.VMEM((max_pairs * 2 * _CHUNK, k), jnp.float32),
            pltpu.VMEM((edges, k), jnp.float32),
            pltpu.VMEM((edges, k), jnp.float32),
            pltpu.VMEM((max_pairs * 2 * edges,), jnp.float32),
            pltpu.SemaphoreType.DMA,
            pltpu.SemaphoreType.DMA,
        ],
    )(functools.partial(_sc_body, n, deg))
    co = sc(qc, d_rows).reshape(n, deg)

    energy = pl.pallas_call(
        _final_body,
        out_shape=jax.ShapeDtypeStruct((n,), jnp.float32),
    )(Z, S, D, co)
    return energy


# speedup vs baseline: 5.3511x; 1.5114x over previous
import functools

import jax
import jax.numpy as jnp
import numpy as np
from jax import lax
from jax.experimental import pallas as pl
from jax.experimental.pallas import tpu as pltpu
from jax.experimental.pallas import tpu_sc as plsc

_LOG_2PI = float(np.log(2.0 * np.pi))

_NC = 2
_NS = 16
_NW = _NC * _NS
_CHUNK = 8


def _softmax_qu(z, s):
    u = 0.5 * (z * z + s * s) + _LOG_2PI
    nu = -u
    m = jnp.max(nu, axis=-1, keepdims=True)
    e = jnp.exp(nu - m)
    q = e / jnp.sum(e, axis=-1, keepdims=True)
    return q, u


def _qc_body(z_ref, s_ref, qc_ref):
    q, _ = _softmax_qu(z_ref[...], s_ref[...])
    qc_ref[...] = jnp.clip(q, 1e-6, 1 - 1e-6)


def _final_body(z_ref, s_ref, d_ref, co_ref, out_ref):
    q, u = _softmax_qu(z_ref[...], s_ref[...])
    e_pt = jnp.sum(q * u, axis=-1)
    pot = -jnp.log(co_ref[...])
    pot = jnp.where(d_ref[...] < 0, 0.0, pot)
    out_ref[...] = e_pt + jnp.sum(pot, axis=-1)


def _sc_body(n_nodes, deg, qc_hbm, d_hbm, co_hbm, idx_v, q_v, neigh0, neigh1,
             co_v, sg0, sg1):
    cid = lax.axis_index("c")
    sid = lax.axis_index("s")
    wid = sid * _NC + cid
    edges = _CHUNK * deg
    n_chunks = n_nodes // _CHUNK
    npt = n_chunks // 2
    lo_p = wid * npt // _NW
    hi_p = (wid + 1) * npt // _NW
    npairs = hi_p - lo_p
    max_pairs = (npt + _NW - 1) // _NW
    lo_r = (lo_p // 2) * 8
    off = lo_p * 4 - lo_r
    pltpu.sync_copy(d_hbm.at[pl.ds(lo_r, max_pairs * 4 + 8)], idx_v)
    pltpu.sync_copy(qc_hbm.at[pl.ds(lo_p * 2 * _CHUNK, max_pairs * 2 * _CHUNK)],
                    q_v)

    lanes = lax.iota(jnp.int32, 16)
    gdn = lax.GatherDimensionNumbers(
        offset_dims=(), collapsed_slice_dims=(0,), start_index_map=(0,))
    perms = [(lanes ^ sh).reshape(16, 1) for sh in (1, 2, 4, 8)]

    def issue(row0, nbuf, sem):
        pltpu.async_copy(qc_hbm.at[idx_v.at[off + row0]],
                         nbuf.at[pl.ds(0, 128)], sem)
        pltpu.async_copy(qc_hbm.at[idx_v.at[off + row0 + 1]],
                         nbuf.at[pl.ds(128, 128)], sem)

    def drain(nbuf, sem):
        pltpu.make_async_copy(qc_hbm.at[idx_v.at[0]],
                              nbuf.at[pl.ds(0, 128)], sem).wait()
        pltpu.make_async_copy(qc_hbm.at[idx_v.at[0]],
                              nbuf.at[pl.ds(128, 128)], sem).wait()

    masks = [(lanes & c) == 0 for c in (1, 2, 4, 8)]

    def permute(x, lvl):
        return lax.gather(x, perms[lvl], gdn, (1,),
                          mode=lax.GatherScatterMode.PROMISE_IN_BOUNDS)

    def compute(cl, nbuf):
        def group_body(grp, carry):
            nloc = cl * _CHUNK + (grp // (deg // 16))
            qs = [q_v[nloc, pl.ds(16 * t, 16)] for t in range(8)]
            base = grp * 16
            ps = []
            for j in range(16):
                e = base + j
                p = nbuf[e, pl.ds(0, 16)] * qs[0]
                for t in range(1, 8):
                    p = p + nbuf[e, pl.ds(16 * t, 16)] * qs[t]
                ps.append(p)
            for lvl in range(4):
                nxt = []
                for a in range(0, len(ps), 2):
                    t0 = ps[a] + permute(ps[a], lvl)
                    t1 = ps[a + 1] + permute(ps[a + 1], lvl)
                    nxt.append(jnp.where(masks[lvl], t0, t1))
                ps = nxt
            co_v[pl.ds(cl * edges + base, 16)] = ps[0]
            return carry

        lax.fori_loop(0, _CHUNK * deg // 16, group_body, 0)

    issue(0, neigh0, sg0)

    def pair_body(p, carry):
        r = p * 4
        issue(r + 2, neigh1, sg1)
        drain(neigh0, sg0)
        compute(2 * p, neigh0)

        @pl.when(p + 1 < npairs)
        def _():
            issue(r + 4, neigh0, sg0)

        drain(neigh1, sg1)
        compute(2 * p + 1, neigh1)
        return carry

    lax.fori_loop(0, npairs, pair_body, 0)
    body_e = (max_pairs - 1) * 2 * edges
    pltpu.sync_copy(co_v.at[pl.ds(0, body_e)],
                    co_hbm.at[pl.ds(lo_p * 2 * edges, body_e)])

    @pl.when(npairs == max_pairs)
    def _():
        pltpu.sync_copy(
            co_v.at[pl.ds(body_e, 2 * edges)],
            co_hbm.at[pl.ds(lo_p * 2 * edges + body_e, 2 * edges)])


def kernel(Z, S, D):
    n, k = Z.shape
    deg = D.shape[1]
    assert k == 128 and n % _CHUNK == 0 and (n * deg) % 128 == 0

    blk = 2000
    grid = n // blk
    qc = pl.pallas_call(
        _qc_body,
        grid=(grid,),
        in_specs=[
            pl.BlockSpec((blk, k), lambda i: (i, 0)),
            pl.BlockSpec((blk, k), lambda i: (i, 0)),
        ],
        out_specs=pl.BlockSpec((blk, k), lambda i: (i, 0)),
        out_shape=jax.ShapeDtypeStruct((n, k), jnp.float32),
    )(Z, S)

    d_rows = jnp.pad(D.reshape(n * deg // 128, 128), ((0, 8), (0, 0)))
    edges = _CHUNK * deg
    npt = (n // _CHUNK) // 2
    max_pairs = (npt + _NW - 1) // _NW
    sc = functools.partial(
        pl.kernel,
        out_type=jax.ShapeDtypeStruct((n * deg,), jnp.float32),
        mesh=plsc.VectorSubcoreMesh(core_axis_name="c", subcore_axis_name="s"),
        scratch_types=[
            pltpu.VMEM((max_pairs * 4 + 8, 128), jnp.int32),
            pltpu.VMEM((max_pairs * 2 * _CHUNK, k), jnp.float32),
            pltpu.VMEM((edges, k), jnp.float32),
            pltpu.VMEM((edges, k), jnp.float32),
            pltpu.VMEM((max_pairs * 2 * edges,), jnp.float32),
            pltpu.SemaphoreType.DMA,
            pltpu.SemaphoreType.DMA,
        ],
    )(functools.partial(_sc_body, n, deg))
    co = sc(qc, d_rows).reshape(n, deg)

    energy = pl.pallas_call(
        _final_body,
        out_shape=jax.ShapeDtypeStruct((n,), jnp.float32),
    )(Z, S, D, co)
    return energy


# speedup vs baseline: 5.4746x; 1.0231x over previous
import functools

import jax
import jax.numpy as jnp
import numpy as np
from jax import lax
from jax.experimental import pallas as pl
from jax.experimental.pallas import tpu as pltpu
from jax.experimental.pallas import tpu_sc as plsc

_LOG_2PI = float(np.log(2.0 * np.pi))

_NC = 2
_NS = 16
_NW = _NC * _NS
_CHUNK = 8


def _softmax_qu(z, s):
    u = 0.5 * (z * z + s * s) + _LOG_2PI
    nu = -u
    m = jnp.max(nu, axis=-1, keepdims=True)
    e = jnp.exp(nu - m)
    q = e / jnp.sum(e, axis=-1, keepdims=True)
    return q, u


def _qc_body(z_ref, s_ref, qc_ref, ept_ref):
    q, u = _softmax_qu(z_ref[...], s_ref[...])
    qc_ref[...] = jnp.clip(q, 1e-6, 1 - 1e-6)
    ept_ref[...] = jnp.sum(q * u, axis=-1, keepdims=True)


def _final_body(ept_ref, d_ref, co_ref, out_ref):
    pot = -jnp.log(co_ref[...])
    pot = jnp.where(d_ref[...] < 0, 0.0, pot)
    out_ref[...] = ept_ref[...][:, 0] + jnp.sum(pot, axis=-1)


def _sc_body(n_nodes, deg, qc_hbm, d_hbm, co_hbm, idx_v, q_v, neigh0, neigh1,
             co_v, sg0, sg1):
    cid = lax.axis_index("c")
    sid = lax.axis_index("s")
    wid = sid * _NC + cid
    edges = _CHUNK * deg
    n_chunks = n_nodes // _CHUNK
    npt = n_chunks // 2
    lo_p = wid * npt // _NW
    hi_p = (wid + 1) * npt // _NW
    npairs = hi_p - lo_p
    max_pairs = (npt + _NW - 1) // _NW
    lo_r = (lo_p // 2) * 8
    off = lo_p * 4 - lo_r
    pltpu.sync_copy(d_hbm.at[pl.ds(lo_r, max_pairs * 4 + 8)], idx_v)
    pltpu.sync_copy(qc_hbm.at[pl.ds(lo_p * 2 * _CHUNK, max_pairs * 2 * _CHUNK)],
                    q_v)

    lanes = lax.iota(jnp.int32, 16)
    gdn = lax.GatherDimensionNumbers(
        offset_dims=(), collapsed_slice_dims=(0,), start_index_map=(0,))
    perms = [(lanes ^ sh).reshape(16, 1) for sh in (1, 2, 4, 8)]

    def issue(row0, nbuf, sem):
        pltpu.async_copy(qc_hbm.at[idx_v.at[off + row0]],
                         nbuf.at[pl.ds(0, 128)], sem)
        pltpu.async_copy(qc_hbm.at[idx_v.at[off + row0 + 1]],
                         nbuf.at[pl.ds(128, 128)], sem)

    def drain(nbuf, sem):
        pltpu.make_async_copy(qc_hbm.at[idx_v.at[0]],
                              nbuf.at[pl.ds(0, 128)], sem).wait()
        pltpu.make_async_copy(qc_hbm.at[idx_v.at[0]],
                              nbuf.at[pl.ds(128, 128)], sem).wait()

    masks = [(lanes & c) == 0 for c in (1, 2, 4, 8)]

    def permute(x, lvl):
        return lax.gather(x, perms[lvl], gdn, (1,),
                          mode=lax.GatherScatterMode.PROMISE_IN_BOUNDS)

    def compute(cl, nbuf):
        def group_body(grp, carry):
            nloc = cl * _CHUNK + (grp // (deg // 16))
            qs = [q_v[nloc, pl.ds(16 * t, 16)] for t in range(8)]
            base = grp * 16
            ps = []
            for j in range(16):
                e = base + j
                p = nbuf[e, pl.ds(0, 16)] * qs[0]
                for t in range(1, 8):
                    p = p + nbuf[e, pl.ds(16 * t, 16)] * qs[t]
                ps.append(p)
            for lvl in range(4):
                nxt = []
                for a in range(0, len(ps), 2):
                    t0 = ps[a] + permute(ps[a], lvl)
                    t1 = ps[a + 1] + permute(ps[a + 1], lvl)
                    nxt.append(jnp.where(masks[lvl], t0, t1))
                ps = nxt
            co_v[pl.ds(cl * edges + base, 16)] = ps[0]
            return carry

        lax.fori_loop(0, _CHUNK * deg // 16, group_body, 0)

    issue(0, neigh0, sg0)

    def pair_body(p, carry):
        r = p * 4
        issue(r + 2, neigh1, sg1)
        drain(neigh0, sg0)
        compute(2 * p, neigh0)

        @pl.when(p + 1 < npairs)
        def _():
            issue(r + 4, neigh0, sg0)

        drain(neigh1, sg1)
        compute(2 * p + 1, neigh1)
        return carry

    lax.fori_loop(0, npairs, pair_body, 0)
    body_e = (max_pairs - 1) * 2 * edges
    pltpu.sync_copy(co_v.at[pl.ds(0, body_e)],
                    co_hbm.at[pl.ds(lo_p * 2 * edges, body_e)])

    @pl.when(npairs == max_pairs)
    def _():
        pltpu.sync_copy(
            co_v.at[pl.ds(body_e, 2 * edges)],
            co_hbm.at[pl.ds(lo_p * 2 * edges + body_e, 2 * edges)])


def kernel(Z, S, D):
    n, k = Z.shape
    deg = D.shape[1]
    assert k == 128 and n % _CHUNK == 0 and (n * deg) % 128 == 0

    blk = 2000
    grid = n // blk
    qc, e_pt = pl.pallas_call(
        _qc_body,
        grid=(grid,),
        in_specs=[
            pl.BlockSpec((blk, k), lambda i: (i, 0)),
            pl.BlockSpec((blk, k), lambda i: (i, 0)),
        ],
        out_specs=[
            pl.BlockSpec((blk, k), lambda i: (i, 0)),
            pl.BlockSpec((blk, 1), lambda i: (i, 0)),
        ],
        out_shape=[
            jax.ShapeDtypeStruct((n, k), jnp.float32),
            jax.ShapeDtypeStruct((n, 1), jnp.float32),
        ],
    )(Z, S)

    d_rows = jnp.pad(D.reshape(n * deg // 128, 128), ((0, 8), (0, 0)))
    edges = _CHUNK * deg
    npt = (n // _CHUNK) // 2
    max_pairs = (npt + _NW - 1) // _NW
    sc = functools.partial(
        pl.kernel,
        out_type=jax.ShapeDtypeStruct((n * deg,), jnp.float32),
        mesh=plsc.VectorSubcoreMesh(core_axis_name="c", subcore_axis_name="s"),
        scratch_types=[
            pltpu.VMEM((max_pairs * 4 + 8, 128), jnp.int32),
            pltpu.VMEM((max_pairs * 2 * _CHUNK, k), jnp.float32),
            pltpu.VMEM((edges, k), jnp.float32),
            pltpu.VMEM((edges, k), jnp.float32),
            pltpu.VMEM((max_pairs * 2 * edges,), jnp.float32),
            pltpu.SemaphoreType.DMA,
            pltpu.SemaphoreType.DMA,
        ],
    )(functools.partial(_sc_body, n, deg))
    co = sc(qc, d_rows).reshape(n, deg)

    energy = pl.pallas_call(
        _final_body,
        out_shape=jax.ShapeDtypeStruct((n,), jnp.float32),
    )(e_pt, D, co)
    return energy


# speedup vs baseline: 6.1732x; 1.1276x over previous
import functools

import jax
import jax.numpy as jnp
import numpy as np
from jax import lax
from jax.experimental import pallas as pl
from jax.experimental.pallas import tpu as pltpu
from jax.experimental.pallas import tpu_sc as plsc

_LOG_2PI = float(np.log(2.0 * np.pi))

_NC = 2
_NS = 16
_NW = _NC * _NS
_CHUNK = 8


def _softmax_qu(z, s):
    u = 0.5 * (z * z + s * s) + _LOG_2PI
    nu = -u
    m = jnp.max(nu, axis=-1, keepdims=True)
    e = jnp.exp(nu - m)
    q = e / jnp.sum(e, axis=-1, keepdims=True)
    return q, u


def _qc_body(z_ref, s_ref, qc_ref, ept_ref):
    q, u = _softmax_qu(z_ref[...], s_ref[...])
    qc_ref[...] = jnp.clip(q, 1e-6, 1 - 1e-6)
    ept_ref[...] = jnp.sum(q * u, axis=-1, keepdims=True)


_LN2 = 0.6931471805599453
_LOG_POLY = (-2.7922169043916756, 8.409026123085447, -14.5952133416544,
             17.84899924612202, -13.688416615872717, 5.919118101828458,
             -1.1012991508085974)


def _neg_log(x):
    ex = jnp.zeros_like(x)
    for kk in (32, 16, 8, 4, 2, 1):
        c = x < (2.0 ** -kk)
        x = jnp.where(c, x * (2.0 ** kk), x)
        ex = ex + jnp.where(c, jnp.float32(kk), 0.0)
    p = jnp.float32(_LOG_POLY[6])
    for c in _LOG_POLY[5::-1]:
        p = p * x + c
    return ex * _LN2 - p


def _sc_body(n_nodes, deg, qc_hbm, d_hbm, ept_hbm, en_hbm, idx_v, q_v, ept_v,
             neigh0, neigh1, en_v, sg0, sg1):
    cid = lax.axis_index("c")
    sid = lax.axis_index("s")
    wid = sid * _NC + cid
    edges = _CHUNK * deg
    n_chunks = n_nodes // _CHUNK
    npt = n_chunks // 2
    lo_p = wid * npt // _NW
    hi_p = (wid + 1) * npt // _NW
    npairs = hi_p - lo_p
    max_pairs = (npt + _NW - 1) // _NW
    lo_r = (lo_p // 2) * 8
    off = lo_p * 4 - lo_r
    pltpu.sync_copy(d_hbm.at[pl.ds(lo_r, max_pairs * 4 + 8)], idx_v)
    pltpu.sync_copy(qc_hbm.at[pl.ds(lo_p * 2 * _CHUNK, max_pairs * 2 * _CHUNK)],
                    q_v)
    pltpu.sync_copy(ept_hbm.at[pl.ds(lo_p * 2 * _CHUNK, max_pairs * 2 * _CHUNK)],
                    ept_v)

    lanes = lax.iota(jnp.int32, 16)
    gdn = lax.GatherDimensionNumbers(
        offset_dims=(), collapsed_slice_dims=(0,), start_index_map=(0,))
    perms = [(lanes ^ sh).reshape(16, 1) for sh in (1, 2, 4, 8)]

    def issue(row0, nbuf, sem):
        pltpu.async_copy(qc_hbm.at[idx_v.at[off + row0]],
                         nbuf.at[pl.ds(0, 128)], sem)
        pltpu.async_copy(qc_hbm.at[idx_v.at[off + row0 + 1]],
                         nbuf.at[pl.ds(128, 128)], sem)

    def drain(nbuf, sem):
        pltpu.make_async_copy(qc_hbm.at[idx_v.at[0]],
                              nbuf.at[pl.ds(0, 128)], sem).wait()
        pltpu.make_async_copy(qc_hbm.at[idx_v.at[0]],
                              nbuf.at[pl.ds(128, 128)], sem).wait()

    masks = [(lanes & c) == 0 for c in (1, 2, 4, 8)]

    def permute(x, lvl):
        return lax.gather(x, perms[lvl], gdn, (1,),
                          mode=lax.GatherScatterMode.PROMISE_IN_BOUNDS)

    def compute(cl, nbuf, half, acc0):
        def group_body(grp, acc):
            nloc = cl * _CHUNK + (grp // (deg // 16))
            qs = [q_v[nloc, pl.ds(16 * t, 16)] for t in range(8)]
            base = grp * 16
            ps = []
            for j in range(16):
                e = base + j
                p = nbuf[e, pl.ds(0, 16)] * qs[0]
                for t in range(1, 8):
                    p = p + nbuf[e, pl.ds(16 * t, 16)] * qs[t]
                ps.append(p)
            for lvl in range(4):
                nxt = []
                for a in range(0, len(ps), 2):
                    t0 = ps[a] + permute(ps[a], lvl)
                    t1 = ps[a + 1] + permute(ps[a + 1], lvl)
                    nxt.append(jnp.where(masks[lvl], t0, t1))
                ps = nxt
            pot = _neg_log(ps[0])
            dvals = idx_v[off + cl * 2 + base // 128,
                          pl.ds(base % 128, 16)]
            pot = jnp.where(dvals < 0, 0.0, pot)
            for lvl in range(4):
                pot = pot + permute(pot, lvl)
            nl = half * _CHUNK + grp // (deg // 16)
            return acc + jnp.where(lanes == nl, pot, 0.0)

        return lax.fori_loop(0, _CHUNK * deg // 16, group_body, acc0)

    issue(0, neigh0, sg0)

    def pair_body(p, carry):
        r = p * 4
        issue(r + 2, neigh1, sg1)
        drain(neigh0, sg0)
        acc = ept_v[pl.ds(p * 16, 16)]
        acc = compute(2 * p, neigh0, 0, acc)

        @pl.when(p + 1 < npairs)
        def _():
            issue(r + 4, neigh0, sg0)

        drain(neigh1, sg1)
        acc = compute(2 * p + 1, neigh1, 1, acc)
        en_v[pl.ds(p * 16, 16)] = acc
        return carry

    lax.fori_loop(0, npairs, pair_body, 0)
    body_n = (max_pairs - 1) * 16
    pltpu.sync_copy(en_v.at[pl.ds(0, body_n)],
                    en_hbm.at[pl.ds(lo_p * 16, body_n)])

    @pl.when(npairs == max_pairs)
    def _():
        pltpu.sync_copy(en_v.at[pl.ds(body_n, 16)],
                        en_hbm.at[pl.ds(lo_p * 16 + body_n, 16)])


def kernel(Z, S, D):
    n, k = Z.shape
    deg = D.shape[1]
    assert k == 128 and n % _CHUNK == 0 and (n * deg) % 128 == 0

    blk = 2000
    grid = n // blk
    qc, e_pt = pl.pallas_call(
        _qc_body,
        grid=(grid,),
        in_specs=[
            pl.BlockSpec((blk, k), lambda i: (i, 0)),
            pl.BlockSpec((blk, k), lambda i: (i, 0)),
        ],
        out_specs=[
            pl.BlockSpec((blk, k), lambda i: (i, 0)),
            pl.BlockSpec((blk, 1), lambda i: (i, 0)),
        ],
        out_shape=[
            jax.ShapeDtypeStruct((n, k), jnp.float32),
            jax.ShapeDtypeStruct((n, 1), jnp.float32),
        ],
    )(Z, S)

    d_rows = jnp.pad(D.reshape(n * deg // 128, 128), ((0, 8), (0, 0)))
    edges = _CHUNK * deg
    npt = (n // _CHUNK) // 2
    max_pairs = (npt + _NW - 1) // _NW
    sc = functools.partial(
        pl.kernel,
        out_type=jax.ShapeDtypeStruct((n,), jnp.float32),
        mesh=plsc.VectorSubcoreMesh(core_axis_name="c", subcore_axis_name="s"),
        scratch_types=[
            pltpu.VMEM((max_pairs * 4 + 8, 128), jnp.int32),
            pltpu.VMEM((max_pairs * 2 * _CHUNK, k), jnp.float32),
            pltpu.VMEM((max_pairs * 2 * _CHUNK,), jnp.float32),
            pltpu.VMEM((edges, k), jnp.float32),
            pltpu.VMEM((edges, k), jnp.float32),
            pltpu.VMEM((max_pairs * 16,), jnp.float32),
            pltpu.SemaphoreType.DMA,
            pltpu.SemaphoreType.DMA,
        ],
    )(functools.partial(_sc_body, n, deg))
    return sc(qc, d_rows, e_pt.reshape(n))
